# baseline (speedup 1.0000x reference)
import functools
import math

import jax
import jax.numpy as jnp
from jax import lax
from jax.experimental import pallas as pl
from jax.experimental.pallas import tpu as pltpu
from jax.experimental.pallas import tpu_sc as plsc

VOCAB = 1000000
EMB = 64
SCALE = math.sqrt(EMB)

_NC = 2
_NS = 16
_NW = _NC * _NS

_CHUNK = 800
_NBUF = 2

_W = 2048
_TB = 8


def _detile_body(tw_ref, out_ref):
    xt = tw_ref[...].T
    out_ref[...] = jnp.concatenate([xt, jnp.zeros_like(xt)], axis=1)


def _detile(tw):
    n_steps = (VOCAB + _W - 1) // _W
    return pl.pallas_call(
        _detile_body,
        grid=(n_steps,),
        in_specs=[pl.BlockSpec((EMB, _W), lambda i: (0, i))],
        out_specs=pl.BlockSpec((_W, 2 * EMB), lambda i: (i, 0)),
        out_shape=jax.ShapeDtypeStruct((VOCAB, 2 * EMB), jnp.float32),
    )(tw)


def _make_gather(B: int):
    per_w = B // _NW
    n_chunks = per_w // _CHUNK
    n_main = n_chunks // _NBUF - 1
    mesh = plsc.VectorSubcoreMesh(core_axis_name="c", subcore_axis_name="s")

    @functools.partial(
        pl.kernel,
        mesh=mesh,
        compiler_params=pltpu.CompilerParams(use_tc_tiling_on_sc=False),
        out_type=jax.ShapeDtypeStruct((B, EMB), jnp.float32),
        scratch_types=[
            pltpu.VMEM((_NBUF, _CHUNK), jnp.int32),
            pltpu.VMEM((_NBUF, _CHUNK, EMB), jnp.float32),
            [pltpu.SemaphoreType.DMA] * _NBUF,
            [pltpu.SemaphoreType.DMA] * _NBUF,
        ],
    )
    def gather_kernel(idx_hbm, table_hbm, out_hbm, idx_v, rows_v, g_sems, o_sems):
        wid = lax.axis_index("s") * _NC + lax.axis_index("c")
        base = wid * per_w

        def start_gather(b, off):
            pltpu.sync_copy(idx_hbm.at[pl.ds(off, _CHUNK)], idx_v.at[b])
            pltpu.async_copy(table_hbm.at[idx_v.at[b]], rows_v.at[b], g_sems[b])

        def wait_gather(b):
            pltpu.make_async_copy(
                table_hbm.at[idx_v.at[b]], rows_v.at[b], g_sems[b]
            ).wait()

        def start_out(b, off):
            pltpu.async_copy(
                rows_v.at[b], out_hbm.at[pl.ds(off, _CHUNK)], o_sems[b]
            )

        def wait_out(b, off):
            pltpu.make_async_copy(
                rows_v.at[b], out_hbm.at[pl.ds(off, _CHUNK)], o_sems[b]
            ).wait()

        for b in range(_NBUF):
            start_gather(b, base + b * _CHUNK)

        def main_body(it, carry):
            off0 = base + it * _NBUF * _CHUNK
            for b in range(_NBUF):
                wait_gather(b)
                start_out(b, off0 + b * _CHUNK)
            for b in range(_NBUF):
                wait_out(b, off0 + b * _CHUNK)
                start_gather(b, off0 + (_NBUF + b) * _CHUNK)
            return carry

        lax.fori_loop(0, n_main, main_body, 0)

        off0 = base + n_main * _NBUF * _CHUNK
        for b in range(_NBUF):
            wait_gather(b)
            start_out(b, off0 + b * _CHUNK)
        for b in range(_NBUF):
            wait_out(b, off0 + b * _CHUNK)

    return gather_kernel


def _retile_body(pp_ref, out_ref):
    x = pp_ref[...]
    xl = x[:, :, 0:EMB]
    xr = x[:, :, EMB:2 * EMB]
    yl = jnp.swapaxes(xl, 1, 2)
    yr = jnp.swapaxes(xr, 1, 2)
    y = jnp.stack([yl, yr], axis=-1).reshape(_TB, EMB, 128)
    y = y * jnp.float32(SCALE)
    out_ref[...] = y.reshape(_TB, 8, 1, 8, 128)


def _retile(pp, T, B4):
    nb = B4 // 128
    return pl.pallas_call(
        _retile_body,
        grid=(T // _TB, nb),
        in_specs=[pl.BlockSpec((_TB, EMB, 128), lambda i, j: (i, j, 0))],
        out_specs=pl.BlockSpec(
            (_TB, 8, 1, 8, 128), lambda i, j: (i, 0, j, 0, 0)
        ),
        out_shape=jax.ShapeDtypeStruct((T, 8, nb, 8, 128), jnp.float32),
    )(pp)


def kernel(tokens, embedding_weight):
    b, t = tokens.shape
    idx_t = (tokens.transpose(1, 0).reshape(-1) * 2).astype(jnp.int32)
    table_lin = _detile(embedding_weight.T).reshape(2 * VOCAB, EMB)
    rows = _make_gather(b * t)(idx_t, table_lin)
    out5 = _retile(rows.reshape(t, b // 2, 2 * EMB), t, b)
    return out5.transpose(2, 4, 0, 1, 3).reshape(b, t, EMB)


# speedup vs baseline: 7.7125x; 7.7125x over previous
import functools
import math

import jax
import jax.numpy as jnp
from jax import lax
from jax.experimental import pallas as pl
from jax.experimental.pallas import tpu as pltpu
from jax.experimental.pallas import tpu_sc as plsc

VOCAB = 1000000
EMB = 64
SCALE = math.sqrt(EMB)

_NC = 2
_NS = 16
_NW = _NC * _NS

_CHUNK = 800
_NBUF = 2

_W = 2048
_TB = 8


def _detile_body(tw_ref, out_ref):
    xt = tw_ref[...].T
    out_ref[...] = jnp.concatenate([xt, jnp.zeros_like(xt)], axis=1)


def _detile(tw):
    n_steps = (VOCAB + _W - 1) // _W
    return pl.pallas_call(
        _detile_body,
        grid=(n_steps,),
        in_specs=[pl.BlockSpec((EMB, _W), lambda i: (0, i))],
        out_specs=pl.BlockSpec((_W, 2 * EMB), lambda i: (i, 0)),
        out_shape=jax.ShapeDtypeStruct((VOCAB, 2 * EMB), jnp.float32),
    )(tw)


def _make_gather(B: int):
    per_w = B // _NW
    n_chunks = per_w // _CHUNK
    n_main = n_chunks // _NBUF - 1
    mesh = plsc.VectorSubcoreMesh(core_axis_name="c", subcore_axis_name="s")

    @functools.partial(
        pl.kernel,
        mesh=mesh,
        compiler_params=pltpu.CompilerParams(use_tc_tiling_on_sc=False),
        out_type=jax.ShapeDtypeStruct((B, EMB), jnp.float32),
        scratch_types=[
            pltpu.VMEM((_NBUF, _CHUNK), jnp.int32),
            pltpu.VMEM((_NBUF, _CHUNK, EMB), jnp.float32),
            [pltpu.SemaphoreType.DMA] * _NBUF,
            [pltpu.SemaphoreType.DMA] * _NBUF,
        ],
    )
    def gather_kernel(idx_hbm, table_hbm, out_hbm, idx_v, rows_v, g_sems, o_sems):
        wid = lax.axis_index("s") * _NC + lax.axis_index("c")
        base = wid * per_w

        def start_gather(b, off):
            pltpu.sync_copy(idx_hbm.at[pl.ds(off, _CHUNK)], idx_v.at[b])
            pltpu.async_copy(table_hbm.at[idx_v.at[b]], rows_v.at[b], g_sems[b])

        def wait_gather(b):
            pltpu.make_async_copy(
                table_hbm.at[idx_v.at[b]], rows_v.at[b], g_sems[b]
            ).wait()

        def start_out(b, off):
            pltpu.async_copy(
                rows_v.at[b], out_hbm.at[pl.ds(off, _CHUNK)], o_sems[b]
            )

        def wait_out(b, off):
            pltpu.make_async_copy(
                rows_v.at[b], out_hbm.at[pl.ds(off, _CHUNK)], o_sems[b]
            ).wait()

        for b in range(_NBUF):
            start_gather(b, base + b * _CHUNK)

        def main_body(it, carry):
            off0 = base + it * _NBUF * _CHUNK
            for b in range(_NBUF):
                wait_gather(b)
                start_out(b, off0 + b * _CHUNK)
            for b in range(_NBUF):
                wait_out(b, off0 + b * _CHUNK)
                start_gather(b, off0 + (_NBUF + b) * _CHUNK)
            return carry

        lax.fori_loop(0, n_main, main_body, 0)

        off0 = base + n_main * _NBUF * _CHUNK
        for b in range(_NBUF):
            wait_gather(b)
            start_out(b, off0 + b * _CHUNK)
        for b in range(_NBUF):
            wait_out(b, off0 + b * _CHUNK)

    return gather_kernel


def _retile_body(pp_ref, out_ref):
    x = pp_ref[...]
    xl = x[:, :, 0:EMB]
    xr = x[:, :, EMB:2 * EMB]
    yl = jnp.swapaxes(xl, 1, 2)
    yr = jnp.swapaxes(xr, 1, 2)
    y = jnp.concatenate([yl, yr], axis=2) * jnp.float32(SCALE)
    out_ref[...] = y.reshape(_TB, 8, 1, 8, 128)


def _retile(pp, T, B4):
    nb = B4 // 128
    return pl.pallas_call(
        _retile_body,
        grid=(T // _TB, nb),
        in_specs=[pl.BlockSpec((_TB, EMB, 128), lambda i, j: (i, j, 0))],
        out_specs=pl.BlockSpec(
            (_TB, 8, 1, 8, 128), lambda i, j: (i, 0, j, 0, 0)
        ),
        out_shape=jax.ShapeDtypeStruct((T, 8, nb, 8, 128), jnp.float32),
    )(pp)


def kernel(tokens, embedding_weight):
    b, t = tokens.shape
    idx_t = tokens.transpose(1, 0).reshape(-1, 2, EMB)
    idx_t = (idx_t.transpose(0, 2, 1).reshape(-1) * 2).astype(jnp.int32)
    table_lin = _detile(embedding_weight.T).reshape(2 * VOCAB, EMB)
    rows = _make_gather(b * t)(idx_t, table_lin)
    out5 = _retile(rows.reshape(t, b // 2, 2 * EMB), t, b)
    return out5.transpose(2, 4, 0, 1, 3).reshape(b, t, EMB)


# speedup vs baseline: 7.7238x; 1.0015x over previous
import functools
import math

import jax
import jax.numpy as jnp
from jax import lax
from jax.experimental import pallas as pl
from jax.experimental.pallas import tpu as pltpu
from jax.experimental.pallas import tpu_sc as plsc

EMB = 64
SCALE = math.sqrt(EMB)

_NC = 2
_NS = 16
_NW = _NC * _NS

_CHUNK = 800
_NBUF = 2


def _make_gather(B: int):
    per_w = B // _NW
    n_chunks = per_w // _CHUNK
    n_main = n_chunks // _NBUF - 1
    mesh = plsc.VectorSubcoreMesh(core_axis_name="c", subcore_axis_name="s")

    @functools.partial(
        pl.kernel,
        mesh=mesh,
        compiler_params=pltpu.CompilerParams(
            use_tc_tiling_on_sc=False,
            skip_device_barrier=True,
            disable_bounds_checks=True,
            disable_semaphore_checks=True,
        ),
        out_type=jax.ShapeDtypeStruct((B, EMB), jnp.float32),
        scratch_types=[
            pltpu.VMEM((_NBUF, _CHUNK), jnp.int32),
            pltpu.VMEM((_NBUF, _CHUNK, EMB), jnp.float32),
            [pltpu.SemaphoreType.DMA] * _NBUF,
            [pltpu.SemaphoreType.DMA] * _NBUF,
        ],
    )
    def gather_kernel(idx_hbm, table_hbm, out_hbm, idx_v, rows_v, g_sems, o_sems):
        wid = lax.axis_index("s") * _NC + lax.axis_index("c")
        base = wid * per_w

        def start_gather(b, off):
            pltpu.sync_copy(idx_hbm.at[pl.ds(off, _CHUNK)], idx_v.at[b])
            pltpu.async_copy(table_hbm.at[idx_v.at[b]], rows_v.at[b], g_sems[b])

        def wait_gather(b):
            pltpu.make_async_copy(
                table_hbm.at[idx_v.at[b]], rows_v.at[b], g_sems[b]
            ).wait()

        def start_out(b, off):
            pltpu.async_copy(
                rows_v.at[b], out_hbm.at[pl.ds(off, _CHUNK)], o_sems[b]
            )

        def wait_out(b, off):
            pltpu.make_async_copy(
                rows_v.at[b], out_hbm.at[pl.ds(off, _CHUNK)], o_sems[b]
            ).wait()

        for b in range(_NBUF):
            start_gather(b, base + b * _CHUNK)

        def main_body(it, carry):
            off0 = base + it * _NBUF * _CHUNK
            for b in range(_NBUF):
                wait_gather(b)
                start_out(b, off0 + b * _CHUNK)
            for b in range(_NBUF):
                wait_out(b, off0 + b * _CHUNK)
                start_gather(b, off0 + (_NBUF + b) * _CHUNK)
            return carry

        lax.fori_loop(0, n_main, main_body, 0)

        off0 = base + n_main * _NBUF * _CHUNK
        for b in range(_NBUF):
            wait_gather(b)
            start_out(b, off0 + b * _CHUNK)
        for b in range(_NBUF):
            wait_out(b, off0 + b * _CHUNK)

    return gather_kernel


def kernel(tokens, embedding_weight):
    b, t = tokens.shape
    flat_idx = tokens.reshape(-1).astype(jnp.int32)
    rows = _make_gather(b * t)(flat_idx, embedding_weight)
    return (rows * jnp.float32(SCALE)).reshape(b, t, EMB)


# speedup vs baseline: 9.3017x; 1.2043x over previous
import functools
import math

import jax
import jax.numpy as jnp
from jax import lax
from jax.experimental import pallas as pl
from jax.experimental.pallas import tpu as pltpu
from jax.experimental.pallas import tpu_sc as plsc

EMB = 64
SCALE = math.sqrt(EMB)

_NC = 2
_NS = 16
_NW = _NC * _NS

_CHUNK = 800
_NBUF = 2


def _make_gather(B: int):
    per_w = B // _NW
    n_chunks = per_w // _CHUNK
    n_main = n_chunks // _NBUF - 1
    mesh = plsc.VectorSubcoreMesh(core_axis_name="c", subcore_axis_name="s")

    @functools.partial(
        pl.kernel,
        mesh=mesh,
        compiler_params=pltpu.CompilerParams(
            use_tc_tiling_on_sc=False,
            skip_device_barrier=True,
            disable_bounds_checks=True,
            disable_semaphore_checks=True,
        ),
        out_type=jax.ShapeDtypeStruct((B, EMB), jnp.float32),
        scratch_types=[
            pltpu.VMEM((_NBUF, _CHUNK), jnp.int32),
            pltpu.VMEM((_NBUF, _CHUNK, EMB), jnp.float32),
            [pltpu.SemaphoreType.DMA] * _NBUF,
            [pltpu.SemaphoreType.DMA] * _NBUF,
        ],
    )
    def gather_kernel(idx_hbm, table_hbm, out_hbm, idx_v, rows_v, g_sems, o_sems):
        wid = lax.axis_index("s") * _NC + lax.axis_index("c")
        base = wid * per_w

        def start_gather(b, off):
            pltpu.sync_copy(idx_hbm.at[pl.ds(off, _CHUNK)], idx_v.at[b])
            pltpu.async_copy(table_hbm.at[idx_v.at[b]], rows_v.at[b], g_sems[b])

        def wait_gather(b):
            pltpu.make_async_copy(
                table_hbm.at[idx_v.at[b]], rows_v.at[b], g_sems[b]
            ).wait()

        def scale_chunk(b):
            def scale_body(i, c):
                for r in range(4):
                    for j in range(EMB // 16):
                        sl = pl.ds(j * 16, 16)
                        row = i * 4 + r
                        rows_v[b, row, sl] = rows_v[b, row, sl] * SCALE
                return c

            lax.fori_loop(0, _CHUNK // 4, scale_body, 0)

        def start_out(b, off):
            pltpu.async_copy(
                rows_v.at[b], out_hbm.at[pl.ds(off, _CHUNK)], o_sems[b]
            )

        def wait_out(b, off):
            pltpu.make_async_copy(
                rows_v.at[b], out_hbm.at[pl.ds(off, _CHUNK)], o_sems[b]
            ).wait()

        for b in range(_NBUF):
            start_gather(b, base + b * _CHUNK)

        def main_body(it, carry):
            off0 = base + it * _NBUF * _CHUNK
            for b in range(_NBUF):
                wait_gather(b)
                scale_chunk(b)
                start_out(b, off0 + b * _CHUNK)
            for b in range(_NBUF):
                wait_out(b, off0 + b * _CHUNK)
                start_gather(b, off0 + (_NBUF + b) * _CHUNK)
            return carry

        lax.fori_loop(0, n_main, main_body, 0)

        off0 = base + n_main * _NBUF * _CHUNK
        for b in range(_NBUF):
            wait_gather(b)
            scale_chunk(b)
            start_out(b, off0 + b * _CHUNK)
        for b in range(_NBUF):
            wait_out(b, off0 + b * _CHUNK)

    return gather_kernel


def kernel(tokens, embedding_weight):
    b, t = tokens.shape
    flat_idx = tokens.reshape(-1).astype(jnp.int32)
    rows = _make_gather(b * t)(flat_idx, embedding_weight)
    return rows.reshape(b, t, EMB)
